# Initial kernel scaffold; baseline (speedup 1.0000x reference)
import jax
import jax.numpy as jnp
from jax.experimental import pallas as pl


def kernel(x, emb_table):
    raise NotImplementedError("write your pallas kernel here")


# speedup vs baseline: 1.6772x; 1.6772x over previous
import jax
import jax.numpy as jnp
from jax.experimental import pallas as pl

BLOCK_S = 1024


def _body(x_ref, t_ref, o_ref):
    o_ref[...] = x_ref[...] + t_ref[...][None]


def kernel(x, emb_table):
    B, S, D = x.shape
    n_s = S // BLOCK_S
    return pl.pallas_call(
        _body,
        grid=(n_s, B),
        in_specs=[
            pl.BlockSpec((1, BLOCK_S, D), lambda i, b: (b, i, 0)),
            pl.BlockSpec((BLOCK_S, D), lambda i, b: (i, 0)),
        ],
        out_specs=pl.BlockSpec((1, BLOCK_S, D), lambda i, b: (b, i, 0)),
        out_shape=jax.ShapeDtypeStruct((B, S, D), x.dtype),
    )(x, emb_table)
